# baseline (speedup 1.0000x reference)
import functools

import jax
import jax.numpy as jnp
from jax import lax
from jax.experimental import pallas as pl
from jax.experimental.pallas import tpu as pltpu
from jax.experimental.pallas import tpu_sc as plsc

D_MODEL = 1024
MAX_SEQ = 8192
SEP_ID = 102
S = 256
WR = S + 16

B = 4
NB = MAX_SEQ // S
WPR = 8
CHUNK = MAX_SEQ // WPR
KB = CHUNK // S


def _splat_sum(vec, red_v):
    idx = lax.iota(jnp.int32, 16)
    for k in (8, 4, 2, 1):
        red_v[...] = vec
        vec = vec + plsc.load_gather(red_v, [jnp.bitwise_xor(idx, k)])
    return vec


def _sc_meta_body(tok_hbm, bases_hbm, counts_hbm,
                  tok_v, allt_v, stage_v, red_v, bst_v, cst_v, totals_sh):
    c = lax.axis_index("c")
    s = lax.axis_index("s")
    row = 2 * c + s // WPR
    cid = s % WPR
    goff = row * MAX_SEQ + cid * CHUNK

    pltpu.sync_copy(tok_hbm.at[pl.ds(goff, CHUNK)], tok_v)

    ones16 = jnp.ones((16,), jnp.int32)
    zeros16 = jnp.zeros((16,), jnp.int32)

    bsum = []
    for k in range(KB):
        acc = zeros16
        for i in range(S // 16):
            v = tok_v[pl.ds((k * (S // 16) + i) * 16, 16)]
            acc = acc + jnp.where(v == SEP_ID, ones16, zeros16)
        bsum.append(_splat_sum(acc, red_v))
    total = bsum[0]
    for k in range(1, KB):
        total = total + bsum[k]

    stage_v[...] = total
    pltpu.sync_copy(stage_v, totals_sh.at[pl.ds(s * 16, 16)])
    plsc.subcore_barrier()
    pltpu.sync_copy(totals_sh, allt_v)
    pref = zeros16
    srow0 = (s // WPR) * WPR
    for i in range(WPR):
        vr = allt_v[pl.ds((srow0 + i) * 16, 16)]
        iv = jnp.full((16,), i, jnp.int32)
        pref = pref + jnp.where(iv < cid, vr, zeros16)

    prev = pref
    for k in range(KB):
        bst_v[pl.ds(k * 16, 16)] = prev
        cst_v[pl.ds(k * 16, 16)] = bsum[k]
        prev = prev + bsum[k]
    moff = (row * NB + cid * KB) * 16
    pltpu.sync_copy(bst_v, bases_hbm.at[pl.ds(moff, KB * 16)])
    pltpu.sync_copy(cst_v, counts_hbm.at[pl.ds(moff, KB * 16)])


def _segment_meta(tokens):
    mesh = plsc.VectorSubcoreMesh(core_axis_name="c", subcore_axis_name="s")
    run = functools.partial(
        pl.kernel,
        out_type=[
            jax.ShapeDtypeStruct((B * NB * 16,), jnp.int32),
            jax.ShapeDtypeStruct((B * NB * 16,), jnp.int32),
        ],
        mesh=mesh,
        compiler_params=pltpu.CompilerParams(needs_layout_passes=False),
        scratch_types=[
            pltpu.VMEM((CHUNK,), jnp.int32),
            pltpu.VMEM((16 * 16,), jnp.int32),
            pltpu.VMEM((16,), jnp.int32),
            pltpu.VMEM((16,), jnp.int32),
            pltpu.VMEM((KB * 16,), jnp.int32),
            pltpu.VMEM((KB * 16,), jnp.int32),
            pltpu.VMEM_SHARED((16 * 16,), jnp.int32),
        ],
    )(_sc_meta_body)
    bases, counts = run(tokens.reshape(-1))
    return bases.reshape(B, NB, 1, 16), counts.reshape(B, NB, 1, 16)


def _main_kernel(x_ref, tok_ref, base_ref, cnt_ref, pe_ref, o_ref):
    b = pl.program_id(0)
    j = pl.program_id(1)
    base = base_ref[b, j]
    cnt = cnt_ref[b, j]
    xb = x_ref[0]

    @pl.when(cnt == 0)
    def _fast():
        p = jnp.minimum(base, MAX_SEQ - 1)
        p8 = (p // 8) * 8
        w8 = pe_ref[pl.ds(p8, 8), :]
        sel = (jax.lax.broadcasted_iota(jnp.int32, (8, 1), 0)
               == (p - p8)).astype(jnp.float32)
        row = jnp.sum(w8 * sel, axis=0, keepdims=True)
        o_ref[0] = xb + row

    @pl.when(cnt != 0)
    def _general():
        m = (tok_ref[0, 0] == SEP_ID).astype(jnp.int32)
        acc = m
        k = 1
        while k < S:
            acc = acc + jnp.concatenate(
                [jnp.zeros((1, k), jnp.int32), acc[:, :-k]], axis=1)
            k *= 2
        pos_v = base + acc
        pos_c = jnp.minimum(pos_v, MAX_SEQ - 1)
        base8 = (jnp.minimum(base, MAX_SEQ - WR) // 8) * 8
        w = pe_ref[pl.ds(base8, WR), :]
        r = pos_c - base8
        oh = (jax.lax.broadcasted_iota(jnp.int32, (WR, S), 0)
              == jnp.broadcast_to(r, (WR, S))).astype(jnp.float32)
        y = jax.lax.dot_general(oh, w, (((0,), (0,)), ((), ())),
                                preferred_element_type=jnp.float32)
        o_ref[0] = xb + y


@jax.jit
def kernel(x, tokens, pe):
    bases, counts = _segment_meta(tokens)
    bases = bases[:, :, 0, 0]
    counts = counts[:, :, 0, 0]
    tok4 = tokens.reshape(B, NB, 1, S)
    out = pl.pallas_call(
        _main_kernel,
        grid=(B, NB),
        in_specs=[
            pl.BlockSpec((1, S, D_MODEL), lambda b, j: (b, j, 0)),
            pl.BlockSpec((1, 1, 1, S), lambda b, j: (b, j, 0, 0)),
            pl.BlockSpec((B, NB), lambda b, j: (0, 0),
                         memory_space=pltpu.SMEM),
            pl.BlockSpec((B, NB), lambda b, j: (0, 0),
                         memory_space=pltpu.SMEM),
            pl.BlockSpec((MAX_SEQ, D_MODEL), lambda b, j: (0, 0)),
        ],
        out_specs=pl.BlockSpec((1, S, D_MODEL), lambda b, j: (b, j, 0)),
        out_shape=jax.ShapeDtypeStruct((B, MAX_SEQ, D_MODEL), jnp.float32),
    )(x, tok4, bases, counts, pe[0])
    return out


# speedup vs baseline: 1.1047x; 1.1047x over previous
import functools

import jax
import jax.numpy as jnp
from jax import lax
from jax.experimental import pallas as pl
from jax.experimental.pallas import tpu as pltpu
from jax.experimental.pallas import tpu_sc as plsc

D_MODEL = 1024
MAX_SEQ = 8192
SEP_ID = 102
S = 256
WR = S + 16

B = 4
NB = MAX_SEQ // S
WPR = 8
CHUNK = MAX_SEQ // WPR
KB = CHUNK // S


def _splat_sum(vec, red_v):
    idx = lax.iota(jnp.int32, 16)
    for k in (8, 4, 2, 1):
        red_v[...] = vec
        vec = vec + plsc.load_gather(red_v, [jnp.bitwise_xor(idx, k)])
    return vec


def _sc_meta_body(tok_hbm, bases_hbm, counts_hbm,
                  tok_v, allt_v, stage_v, red_v, bst_v, cst_v, totals_sh):
    c = lax.axis_index("c")
    s = lax.axis_index("s")
    row = 2 * c + s // WPR
    cid = s % WPR
    goff = row * MAX_SEQ + cid * CHUNK

    pltpu.sync_copy(tok_hbm.at[pl.ds(goff, CHUNK)], tok_v)

    ones16 = jnp.ones((16,), jnp.int32)
    zeros16 = jnp.zeros((16,), jnp.int32)

    bsum = []
    for k in range(KB):
        acc = zeros16
        for i in range(S // 16):
            v = tok_v[pl.ds((k * (S // 16) + i) * 16, 16)]
            acc = acc + jnp.where(v == SEP_ID, ones16, zeros16)
        bsum.append(_splat_sum(acc, red_v))
    total = bsum[0]
    for k in range(1, KB):
        total = total + bsum[k]

    stage_v[...] = total
    pltpu.sync_copy(stage_v, totals_sh.at[pl.ds(s * 16, 16)])
    plsc.subcore_barrier()
    pltpu.sync_copy(totals_sh, allt_v)
    pref = zeros16
    srow0 = (s // WPR) * WPR
    for i in range(WPR):
        vr = allt_v[pl.ds((srow0 + i) * 16, 16)]
        iv = jnp.full((16,), i, jnp.int32)
        pref = pref + jnp.where(iv < cid, vr, zeros16)

    prev = pref
    for k in range(KB):
        bst_v[pl.ds(k * 16, 16)] = prev
        cst_v[pl.ds(k * 16, 16)] = bsum[k]
        prev = prev + bsum[k]
    moff = (row * NB + cid * KB) * 16
    pltpu.sync_copy(bst_v, bases_hbm.at[pl.ds(moff, KB * 16)])
    pltpu.sync_copy(cst_v, counts_hbm.at[pl.ds(moff, KB * 16)])


def _segment_meta(tokens):
    mesh = plsc.VectorSubcoreMesh(core_axis_name="c", subcore_axis_name="s")
    run = functools.partial(
        pl.kernel,
        out_type=[
            jax.ShapeDtypeStruct((B * NB * 16,), jnp.int32),
            jax.ShapeDtypeStruct((B * NB * 16,), jnp.int32),
        ],
        mesh=mesh,
        compiler_params=pltpu.CompilerParams(needs_layout_passes=False),
        scratch_types=[
            pltpu.VMEM((CHUNK,), jnp.int32),
            pltpu.VMEM((16 * 16,), jnp.int32),
            pltpu.VMEM((16,), jnp.int32),
            pltpu.VMEM((16,), jnp.int32),
            pltpu.VMEM((KB * 16,), jnp.int32),
            pltpu.VMEM((KB * 16,), jnp.int32),
            pltpu.VMEM_SHARED((16 * 16,), jnp.int32),
        ],
    )(_sc_meta_body)
    bases, counts = run(tokens.reshape(-1))
    return bases.reshape(B, NB, 1, 16), counts.reshape(B, NB, 1, 16)


def _main_kernel(x_ref, tok_ref, base_ref, cnt_ref, pe_ref, o_ref):
    b = pl.program_id(0)
    j = pl.program_id(1)
    base = base_ref[b, j]
    cnt = cnt_ref[b, j]
    xb = x_ref[0]

    @pl.when(cnt == 0)
    def _fast():
        p = jnp.minimum(base, MAX_SEQ - 1)
        p8 = (p // 8) * 8
        w8 = pe_ref[pl.ds(p8, 8), :]
        sel = (jax.lax.broadcasted_iota(jnp.int32, (8, 1), 0)
               == (p - p8)).astype(jnp.float32)
        row = jnp.sum(w8 * sel, axis=0, keepdims=True)
        o_ref[0] = xb + row

    @pl.when(cnt != 0)
    def _general():
        m = (tok_ref[0, 0] == SEP_ID).astype(jnp.int32)
        acc = m
        k = 1
        while k < S:
            acc = acc + jnp.concatenate(
                [jnp.zeros((1, k), jnp.int32), acc[:, :-k]], axis=1)
            k *= 2
        pos_v = base + acc
        pos_c = jnp.minimum(pos_v, MAX_SEQ - 1)
        base8 = (jnp.minimum(base, MAX_SEQ - WR) // 8) * 8
        w = pe_ref[pl.ds(base8, WR), :]
        r = pos_c - base8
        oh = (jax.lax.broadcasted_iota(jnp.int32, (WR, S), 0)
              == jnp.broadcast_to(r, (WR, S))).astype(jnp.float32)
        y = jax.lax.dot_general(oh, w, (((0,), (0,)), ((), ())),
                                preferred_element_type=jnp.float32)
        o_ref[0] = xb + y


def _pos_kernel_tc(tok_ref, pos_ref):
    m = (tok_ref[...] == SEP_ID).astype(jnp.int32)
    acc = m
    k = 1
    while k < MAX_SEQ:
        zeros = jnp.zeros((acc.shape[0], k), jnp.int32)
        acc = acc + jnp.concatenate([zeros, acc[:, :-k]], axis=1)
        k *= 2
    pos_ref[...] = acc


def _segment_meta_tc(tokens):
    pos = pl.pallas_call(
        _pos_kernel_tc,
        out_shape=jax.ShapeDtypeStruct((B, MAX_SEQ), jnp.int32),
    )(tokens)
    ends = pos.reshape(B, NB, S)[:, :, -1]
    bases = jnp.concatenate(
        [jnp.zeros((B, 1), jnp.int32), ends[:, :-1]], axis=1)
    counts = ends - bases
    return bases, counts


@jax.jit
def kernel(x, tokens, pe):
    bases, counts = _segment_meta_tc(tokens)
    tok4 = tokens.reshape(B, NB, 1, S)
    out = pl.pallas_call(
        _main_kernel,
        grid=(B, NB),
        in_specs=[
            pl.BlockSpec((1, S, D_MODEL), lambda b, j: (b, j, 0)),
            pl.BlockSpec((1, 1, 1, S), lambda b, j: (b, j, 0, 0)),
            pl.BlockSpec((B, NB), lambda b, j: (0, 0),
                         memory_space=pltpu.SMEM),
            pl.BlockSpec((B, NB), lambda b, j: (0, 0),
                         memory_space=pltpu.SMEM),
            pl.BlockSpec((MAX_SEQ, D_MODEL), lambda b, j: (0, 0)),
        ],
        out_specs=pl.BlockSpec((1, S, D_MODEL), lambda b, j: (b, j, 0)),
        out_shape=jax.ShapeDtypeStruct((B, MAX_SEQ, D_MODEL), jnp.float32),
    )(x, tok4, bases, counts, pe[0])
    return out


# speedup vs baseline: 1.3578x; 1.2291x over previous
import functools

import jax
import jax.numpy as jnp
from jax import lax
from jax.experimental import pallas as pl
from jax.experimental.pallas import tpu as pltpu
from jax.experimental.pallas import tpu_sc as plsc

D_MODEL = 1024
MAX_SEQ = 8192
SEP_ID = 102
S = 512
WR = S + 16

B = 4
NB = MAX_SEQ // S
WPR = 8
CHUNK = MAX_SEQ // WPR
KB = CHUNK // S


def _splat_sum(vec, red_v):
    idx = lax.iota(jnp.int32, 16)
    for k in (8, 4, 2, 1):
        red_v[...] = vec
        vec = vec + plsc.load_gather(red_v, [jnp.bitwise_xor(idx, k)])
    return vec


def _sc_meta_body(tok_hbm, bases_hbm, counts_hbm,
                  tok_v, allt_v, stage_v, red_v, bst_v, cst_v, totals_sh):
    c = lax.axis_index("c")
    s = lax.axis_index("s")
    row = 2 * c + s // WPR
    cid = s % WPR
    goff = row * MAX_SEQ + cid * CHUNK

    pltpu.sync_copy(tok_hbm.at[pl.ds(goff, CHUNK)], tok_v)

    ones16 = jnp.ones((16,), jnp.int32)
    zeros16 = jnp.zeros((16,), jnp.int32)

    bsum = []
    for k in range(KB):
        acc = zeros16
        for i in range(S // 16):
            v = tok_v[pl.ds((k * (S // 16) + i) * 16, 16)]
            acc = acc + jnp.where(v == SEP_ID, ones16, zeros16)
        bsum.append(_splat_sum(acc, red_v))
    total = bsum[0]
    for k in range(1, KB):
        total = total + bsum[k]

    stage_v[...] = total
    pltpu.sync_copy(stage_v, totals_sh.at[pl.ds(s * 16, 16)])
    plsc.subcore_barrier()
    pltpu.sync_copy(totals_sh, allt_v)
    pref = zeros16
    srow0 = (s // WPR) * WPR
    for i in range(WPR):
        vr = allt_v[pl.ds((srow0 + i) * 16, 16)]
        iv = jnp.full((16,), i, jnp.int32)
        pref = pref + jnp.where(iv < cid, vr, zeros16)

    prev = pref
    for k in range(KB):
        bst_v[pl.ds(k * 16, 16)] = prev
        cst_v[pl.ds(k * 16, 16)] = bsum[k]
        prev = prev + bsum[k]
    moff = (row * NB + cid * KB) * 16
    pltpu.sync_copy(bst_v, bases_hbm.at[pl.ds(moff, KB * 16)])
    pltpu.sync_copy(cst_v, counts_hbm.at[pl.ds(moff, KB * 16)])


def _segment_meta(tokens):
    mesh = plsc.VectorSubcoreMesh(core_axis_name="c", subcore_axis_name="s")
    run = functools.partial(
        pl.kernel,
        out_type=[
            jax.ShapeDtypeStruct((B * NB * 16,), jnp.int32),
            jax.ShapeDtypeStruct((B * NB * 16,), jnp.int32),
        ],
        mesh=mesh,
        compiler_params=pltpu.CompilerParams(needs_layout_passes=False),
        scratch_types=[
            pltpu.VMEM((CHUNK,), jnp.int32),
            pltpu.VMEM((16 * 16,), jnp.int32),
            pltpu.VMEM((16,), jnp.int32),
            pltpu.VMEM((16,), jnp.int32),
            pltpu.VMEM((KB * 16,), jnp.int32),
            pltpu.VMEM((KB * 16,), jnp.int32),
            pltpu.VMEM_SHARED((16 * 16,), jnp.int32),
        ],
    )(_sc_meta_body)
    bases, counts = run(tokens.reshape(-1))
    return bases.reshape(B, NB, 1, 16), counts.reshape(B, NB, 1, 16)


def _main_kernel(x_ref, tok_ref, base_ref, cnt_ref, pe_ref, o_ref):
    b = pl.program_id(0)
    j = pl.program_id(1)
    base = base_ref[b, j]
    cnt = cnt_ref[b, j]
    xb = x_ref[0]

    @pl.when(cnt == 0)
    def _fast():
        p = jnp.minimum(base, MAX_SEQ - 1)
        p8 = (p // 8) * 8
        w8 = pe_ref[pl.ds(p8, 8), :]
        sel = (jax.lax.broadcasted_iota(jnp.int32, (8, 1), 0)
               == (p - p8)).astype(jnp.float32)
        row = jnp.sum(w8 * sel, axis=0, keepdims=True)
        o_ref[0] = xb + row

    @pl.when(cnt != 0)
    def _general():
        m = (tok_ref[0, 0] == SEP_ID).astype(jnp.int32)
        acc = m
        k = 1
        while k < S:
            acc = acc + jnp.concatenate(
                [jnp.zeros((1, k), jnp.int32), acc[:, :-k]], axis=1)
            k *= 2
        pos_v = base + acc
        pos_c = jnp.minimum(pos_v, MAX_SEQ - 1)
        base8 = (jnp.minimum(base, MAX_SEQ - WR) // 8) * 8
        w = pe_ref[pl.ds(base8, WR), :]
        r = pos_c - base8
        oh = (jax.lax.broadcasted_iota(jnp.int32, (WR, S), 0)
              == jnp.broadcast_to(r, (WR, S))).astype(jnp.float32)
        y = jax.lax.dot_general(oh, w, (((0,), (0,)), ((), ())),
                                preferred_element_type=jnp.float32)
        o_ref[0] = xb + y


def _pos_kernel_tc(tok_ref, pos_ref):
    m = (tok_ref[...] == SEP_ID).astype(jnp.int32)
    acc = m
    k = 1
    while k < MAX_SEQ:
        zeros = jnp.zeros((acc.shape[0], k), jnp.int32)
        acc = acc + jnp.concatenate([zeros, acc[:, :-k]], axis=1)
        k *= 2
    pos_ref[...] = acc


def _segment_meta_tc(tokens):
    pos = pl.pallas_call(
        _pos_kernel_tc,
        out_shape=jax.ShapeDtypeStruct((B, MAX_SEQ), jnp.int32),
    )(tokens)
    ends = pos.reshape(B, NB, S)[:, :, -1]
    bases = jnp.concatenate(
        [jnp.zeros((B, 1), jnp.int32), ends[:, :-1]], axis=1)
    counts = ends - bases
    return bases, counts


@jax.jit
def kernel(x, tokens, pe):
    bases, counts = _segment_meta_tc(tokens)
    tok4 = tokens.reshape(B, NB, 1, S)
    out = pl.pallas_call(
        _main_kernel,
        grid=(B, NB),
        in_specs=[
            pl.BlockSpec((1, S, D_MODEL), lambda b, j: (b, j, 0)),
            pl.BlockSpec((1, 1, 1, S), lambda b, j: (b, j, 0, 0)),
            pl.BlockSpec((B, NB), lambda b, j: (0, 0),
                         memory_space=pltpu.SMEM),
            pl.BlockSpec((B, NB), lambda b, j: (0, 0),
                         memory_space=pltpu.SMEM),
            pl.BlockSpec((MAX_SEQ, D_MODEL), lambda b, j: (0, 0)),
        ],
        out_specs=pl.BlockSpec((1, S, D_MODEL), lambda b, j: (b, j, 0)),
        out_shape=jax.ShapeDtypeStruct((B, MAX_SEQ, D_MODEL), jnp.float32),
    )(x, tok4, bases, counts, pe[0])
    return out


# speedup vs baseline: 1.7502x; 1.2890x over previous
import functools

import jax
import jax.numpy as jnp
from jax import lax
from jax.experimental import pallas as pl
from jax.experimental.pallas import tpu as pltpu
from jax.experimental.pallas import tpu_sc as plsc

D_MODEL = 1024
MAX_SEQ = 8192
SEP_ID = 102
S = 1024
WR = S + 16
WRS = 32

B = 4
NB = MAX_SEQ // S
WPR = 8
CHUNK = MAX_SEQ // WPR
KB = CHUNK // S


def _splat_sum(vec, red_v):
    idx = lax.iota(jnp.int32, 16)
    for k in (8, 4, 2, 1):
        red_v[...] = vec
        vec = vec + plsc.load_gather(red_v, [jnp.bitwise_xor(idx, k)])
    return vec


def _sc_meta_body(tok_hbm, bases_hbm, counts_hbm,
                  tok_v, allt_v, stage_v, red_v, bst_v, cst_v, totals_sh):
    c = lax.axis_index("c")
    s = lax.axis_index("s")
    row = 2 * c + s // WPR
    cid = s % WPR
    goff = row * MAX_SEQ + cid * CHUNK

    pltpu.sync_copy(tok_hbm.at[pl.ds(goff, CHUNK)], tok_v)

    ones16 = jnp.ones((16,), jnp.int32)
    zeros16 = jnp.zeros((16,), jnp.int32)

    bsum = []
    for k in range(KB):
        acc = zeros16
        for i in range(S // 16):
            v = tok_v[pl.ds((k * (S // 16) + i) * 16, 16)]
            acc = acc + jnp.where(v == SEP_ID, ones16, zeros16)
        bsum.append(_splat_sum(acc, red_v))
    total = bsum[0]
    for k in range(1, KB):
        total = total + bsum[k]

    stage_v[...] = total
    pltpu.sync_copy(stage_v, totals_sh.at[pl.ds(s * 16, 16)])
    plsc.subcore_barrier()
    pltpu.sync_copy(totals_sh, allt_v)
    pref = zeros16
    srow0 = (s // WPR) * WPR
    for i in range(WPR):
        vr = allt_v[pl.ds((srow0 + i) * 16, 16)]
        iv = jnp.full((16,), i, jnp.int32)
        pref = pref + jnp.where(iv < cid, vr, zeros16)

    prev = pref
    for k in range(KB):
        bst_v[pl.ds(k * 16, 16)] = prev
        cst_v[pl.ds(k * 16, 16)] = bsum[k]
        prev = prev + bsum[k]
    moff = (row * NB + cid * KB) * 16
    pltpu.sync_copy(bst_v, bases_hbm.at[pl.ds(moff, KB * 16)])
    pltpu.sync_copy(cst_v, counts_hbm.at[pl.ds(moff, KB * 16)])


def _segment_meta(tokens):
    mesh = plsc.VectorSubcoreMesh(core_axis_name="c", subcore_axis_name="s")
    run = functools.partial(
        pl.kernel,
        out_type=[
            jax.ShapeDtypeStruct((B * NB * 16,), jnp.int32),
            jax.ShapeDtypeStruct((B * NB * 16,), jnp.int32),
        ],
        mesh=mesh,
        compiler_params=pltpu.CompilerParams(needs_layout_passes=False),
        scratch_types=[
            pltpu.VMEM((CHUNK,), jnp.int32),
            pltpu.VMEM((16 * 16,), jnp.int32),
            pltpu.VMEM((16,), jnp.int32),
            pltpu.VMEM((16,), jnp.int32),
            pltpu.VMEM((KB * 16,), jnp.int32),
            pltpu.VMEM((KB * 16,), jnp.int32),
            pltpu.VMEM_SHARED((16 * 16,), jnp.int32),
        ],
    )(_sc_meta_body)
    bases, counts = run(tokens.reshape(-1))
    return bases.reshape(B, NB, 1, 16), counts.reshape(B, NB, 1, 16)


def _main_kernel(x_ref, tok_ref, base_ref, cnt_ref, pe_ref, o_ref):
    b = pl.program_id(0)
    j = pl.program_id(1)
    base = base_ref[b, j]
    cnt = cnt_ref[b, j]
    xb = x_ref[0]

    @pl.when(cnt == 0)
    def _fast():
        p = jnp.minimum(base, MAX_SEQ - 1)
        p8 = (p // 8) * 8
        w8 = pe_ref[pl.ds(p8, 8), :]
        sel = (jax.lax.broadcasted_iota(jnp.int32, (8, 1), 0)
               == (p - p8)).astype(jnp.float32)
        row = jnp.sum(w8 * sel, axis=0, keepdims=True)
        o_ref[0] = xb + row

    def _window_gather(nrows):
        m = (tok_ref[0, 0] == SEP_ID).astype(jnp.int32)
        acc = m
        k = 1
        while k < S:
            acc = acc + jnp.concatenate(
                [jnp.zeros((1, k), jnp.int32), acc[:, :-k]], axis=1)
            k *= 2
        pos_v = base + acc
        pos_c = jnp.minimum(pos_v, MAX_SEQ - 1)
        base8 = (jnp.minimum(base, MAX_SEQ - nrows) // 8) * 8
        w = pe_ref[pl.ds(base8, nrows), :]
        r = pos_c - base8
        oh = (jax.lax.broadcasted_iota(jnp.int32, (nrows, S), 0)
              == jnp.broadcast_to(r, (nrows, S))).astype(jnp.float32)
        y = jax.lax.dot_general(oh, w, (((0,), (0,)), ((), ())),
                                preferred_element_type=jnp.float32)
        o_ref[0] = xb + y

    @pl.when(jnp.logical_and(cnt > 0, cnt <= WRS - 8))
    def _small():
        _window_gather(WRS)

    @pl.when(cnt > WRS - 8)
    def _full():
        _window_gather(WR)


def _pos_kernel_tc(tok_ref, pos_ref):
    m = (tok_ref[...] == SEP_ID).astype(jnp.int32)
    acc = m
    k = 1
    while k < MAX_SEQ:
        zeros = jnp.zeros((acc.shape[0], k), jnp.int32)
        acc = acc + jnp.concatenate([zeros, acc[:, :-k]], axis=1)
        k *= 2
    pos_ref[...] = acc


def _segment_meta_tc(tokens):
    pos = pl.pallas_call(
        _pos_kernel_tc,
        out_shape=jax.ShapeDtypeStruct((B, MAX_SEQ), jnp.int32),
    )(tokens)
    ends = pos.reshape(B, NB, S)[:, :, -1]
    bases = jnp.concatenate(
        [jnp.zeros((B, 1), jnp.int32), ends[:, :-1]], axis=1)
    counts = ends - bases
    return bases, counts


@jax.jit
def kernel(x, tokens, pe):
    bases, counts = _segment_meta_tc(tokens)
    tok4 = tokens.reshape(B, NB, 1, S)
    out = pl.pallas_call(
        _main_kernel,
        grid=(B, NB),
        in_specs=[
            pl.BlockSpec((1, S, D_MODEL), lambda b, j: (b, j, 0)),
            pl.BlockSpec((1, 1, 1, S), lambda b, j: (b, j, 0, 0)),
            pl.BlockSpec((B, NB), lambda b, j: (0, 0),
                         memory_space=pltpu.SMEM),
            pl.BlockSpec((B, NB), lambda b, j: (0, 0),
                         memory_space=pltpu.SMEM),
            pl.BlockSpec((MAX_SEQ, D_MODEL), lambda b, j: (0, 0)),
        ],
        out_specs=pl.BlockSpec((1, S, D_MODEL), lambda b, j: (b, j, 0)),
        out_shape=jax.ShapeDtypeStruct((B, MAX_SEQ, D_MODEL), jnp.float32),
    )(x, tok4, bases, counts, pe[0])
    return out
